# baseline (speedup 1.0000x reference)
import functools

import jax
import jax.numpy as jnp
from jax import lax
from jax.experimental import pallas as pl
from jax.experimental.pallas import tpu as pltpu
from jax.experimental.pallas import tpu_sc as plsc

B, L = 4096, 200
EMBED = 8
OUT_D = 3 * EMBED

NC, NS = 2, 16
NW = NC * NS
BT = B // NW
LC = 8
M = LC * BT
NCH = L // LC
TIME_V = 1442
DIS_V = 100000
SPEED_V = 1002


def _body(t_hbm, d_hbm, s_hbm, wtT_hbm, wd_hbm, wsT_hbm, out_hbm,
          ti_v, di_v, si_v, gb_v, ob_v, wtT_v, wsT_v, ssem, gsem, osem):
    sid = lax.axis_index("s")
    wid = sid * NC + lax.axis_index("c")
    b0 = wid * BT
    iota = lax.iota(jnp.int32, 16)
    seconst = [jnp.full((16,), se, jnp.int32) for se in range(EMBED)]

    pltpu.sync_copy(wtT_hbm, wtT_v)
    pltpu.sync_copy(wsT_hbm, wsT_v)

    def stage(p, k):
        for idx_hbm, idx_v in ((t_hbm, ti_v), (s_hbm, si_v)):
            pltpu.async_copy(idx_hbm.at[p, wid], idx_v.at[k], ssem)
        for lp in range(LC):
            pltpu.async_copy(
                d_hbm.at[p, wid, lp],
                di_v.at[k, pl.ds(lp * BT, BT)], ssem)

    def stage_wait(k):
        for _ in range(3):
            pltpu.make_async_copy(
                t_hbm.at[0, 0], ti_v.at[k], ssem).wait()

    def gathers(p, k):
        pltpu.async_copy(wd_hbm.at[di_v.at[k]], gb_v.at[k], gsem)

    def gathers_wait(k):
        pltpu.make_async_copy(
            wd_hbm.at[di_v.at[k]], gb_v.at[k], gsem).wait()

    def transpose(p, k):
        @plsc.parallel_loop(0, LC * (BT // 16), 1, unroll=32)
        def per_blk(i):
            lp = i // (BT // 16)
            sbg = lax.rem(i, BT // 16)
            tvi = ti_v[k, lp, pl.ds(sbg * 16, 16)]
            svi = si_v[k, lp, pl.ds(sbg * 16, 16)]
            rows = iota + i * 16
            for se in range(EMBED):
                vt = plsc.load_gather(wtT_v, [seconst[se], tvi])
                ob_v[k, lp, 0, se, pl.ds(sbg * 16, 16)] = vt
                vd = plsc.load_gather(gb_v.at[k], [rows, seconst[se]])
                ob_v[k, lp, 1, se, pl.ds(sbg * 16, 16)] = vd
                vs = plsc.load_gather(wsT_v, [seconst[se], svi])
                ob_v[k, lp, 2, se, pl.ds(sbg * 16, 16)] = vs

    def write(p, k):
        pltpu.async_copy(
            ob_v.at[k], out_hbm.at[pl.ds(p * LC, LC), :, wid], osem)

    def write_wait(k):
        pltpu.make_async_copy(
            ob_v.at[k], out_hbm.at[pl.ds(0, LC), :, wid], osem).wait()

    stage(0, 0)
    stage(1, 1)
    stage_wait(0)
    gathers(0, 0)

    def chunk_body(p, _):
        for kk in (0, 1):

            @pl.when(lax.rem(p, 2) == kk)
            def _(kk=kk):
                @pl.when(p + 1 < NCH)
                def _():
                    stage_wait(kk ^ 1)

                gathers_wait(kk)

                @pl.when(p + 1 < NCH)
                def _():
                    gathers(p + 1, kk ^ 1)

                @pl.when(p >= 2)
                def _():
                    write_wait(kk)

                transpose(p, kk)

                @pl.when(p + 2 < NCH)
                def _():
                    stage(p + 2, kk)

                write(p, kk)

        return 0

    lax.fori_loop(0, NCH, chunk_body, 0)
    write_wait(0 if NCH % 2 == 1 else 1)
    write_wait(1 if NCH % 2 == 1 else 0)


@functools.partial(jax.jit, static_argnames=())
def kernel(time, dis, speed, W_time, W_dis, W_speed):
    def tiled(x):
        return (x.astype(jnp.int32).T
                .reshape(L // 8, 8, NW, BT).transpose(0, 2, 1, 3))

    t = tiled(time)
    d = tiled(dis)
    s = tiled(speed)

    mesh = plsc.VectorSubcoreMesh(core_axis_name="c", subcore_axis_name="s")
    run = pl.kernel(
        _body,
        out_type=jax.ShapeDtypeStruct((L, 3, NW, EMBED, BT), jnp.float32),
        mesh=mesh,
        scratch_types=[
            pltpu.VMEM((2, LC, BT), jnp.int32),
            pltpu.VMEM((2, M), jnp.int32),
            pltpu.VMEM((2, LC, BT), jnp.int32),
            pltpu.VMEM((2, M, EMBED), jnp.float32),
            pltpu.VMEM((2, LC, 3, EMBED, BT), jnp.float32),
            pltpu.VMEM((EMBED, TIME_V), jnp.float32),
            pltpu.VMEM((EMBED, SPEED_V), jnp.float32),
            pltpu.SemaphoreType.DMA,
            pltpu.SemaphoreType.DMA,
            pltpu.SemaphoreType.DMA,
        ],
        compiler_params=pltpu.CompilerParams(
            needs_layout_passes=False,
            use_tc_tiling_on_sc=False,
        ),
    )
    out5 = run(t, d, s, W_time.T, W_dis, W_speed.T)
    return out5.transpose(2, 4, 0, 1, 3).reshape(B, L, OUT_D)


# speedup vs baseline: 1.0988x; 1.0988x over previous
import functools

import jax
import jax.numpy as jnp
from jax import lax
from jax.experimental import pallas as pl
from jax.experimental.pallas import tpu as pltpu
from jax.experimental.pallas import tpu_sc as plsc

B, L = 4096, 200
EMBED = 8
OUT_D = 3 * EMBED

NC, NS = 2, 16
NW = NC * NS
BT = B // NW
LC = 8
M = LC * BT
NCH = L // LC
TIME_V = 1442
DIS_V = 100000
SPEED_V = 1002


def _body(t_hbm, d_hbm, s_hbm, wtT_hbm, wd_hbm, wsT_hbm, out_hbm,
          ti_v, di_v, si_v, gb_v, ob_v, wtT_v, wsT_v, ssem, gsem, osem):
    sid = lax.axis_index("s")
    wid = sid * NC + lax.axis_index("c")
    b0 = wid * BT
    iota = lax.iota(jnp.int32, 16)
    seconst = [jnp.full((16,), se, jnp.int32) for se in range(EMBED)]

    pltpu.sync_copy(wtT_hbm, wtT_v)
    pltpu.sync_copy(wsT_hbm, wsT_v)

    def stage(p, k):
        for idx_hbm, idx_v in ((t_hbm, ti_v), (s_hbm, si_v)):
            pltpu.async_copy(idx_hbm.at[p, wid], idx_v.at[k], ssem)
        for lp in range(LC):
            pltpu.async_copy(
                d_hbm.at[p, wid, lp],
                di_v.at[k, pl.ds(lp * BT, BT)], ssem)

    def stage_wait(k):
        for _ in range(3):
            pltpu.make_async_copy(
                t_hbm.at[0, 0], ti_v.at[k], ssem).wait()

    def gathers(p, k):
        pltpu.async_copy(wd_hbm.at[di_v.at[k]], gb_v.at[k], gsem)

    def gathers_wait(k):
        pltpu.make_async_copy(
            wd_hbm.at[di_v.at[k]], gb_v.at[k], gsem).wait()

    def transpose(p, k):
        @plsc.parallel_loop(0, LC * (BT // 16), 1, unroll=16)
        def per_blk(i):
            lp = i // (BT // 16)
            sbg = lax.rem(i, BT // 16)
            tvi = ti_v[k, lp, pl.ds(sbg * 16, 16)]
            svi = si_v[k, lp, pl.ds(sbg * 16, 16)]
            rows = iota + i * 16
            for se in range(EMBED):
                vt = plsc.load_gather(wtT_v, [seconst[se], tvi])
                ob_v[k, lp, 0, se, pl.ds(sbg * 16, 16)] = vt
                vd = plsc.load_gather(gb_v.at[k], [rows, seconst[se]])
                ob_v[k, lp, 1, se, pl.ds(sbg * 16, 16)] = vd
                vs = plsc.load_gather(wsT_v, [seconst[se], svi])
                ob_v[k, lp, 2, se, pl.ds(sbg * 16, 16)] = vs

    def write(p, k):
        pltpu.async_copy(
            ob_v.at[k], out_hbm.at[pl.ds(p * LC, LC), :, wid], osem)

    def write_wait(k):
        pltpu.make_async_copy(
            ob_v.at[k], out_hbm.at[pl.ds(0, LC), :, wid], osem).wait()

    stage(0, 0)
    stage(1, 1)
    stage_wait(0)
    gathers(0, 0)

    def chunk_body(p, _):
        for kk in (0, 1):

            @pl.when(lax.rem(p, 2) == kk)
            def _(kk=kk):
                @pl.when(p + 1 < NCH)
                def _():
                    stage_wait(kk ^ 1)

                gathers_wait(kk)

                @pl.when(p + 1 < NCH)
                def _():
                    gathers(p + 1, kk ^ 1)

                @pl.when(p >= 2)
                def _():
                    write_wait(kk)

                transpose(p, kk)

                @pl.when(p + 2 < NCH)
                def _():
                    stage(p + 2, kk)

                write(p, kk)

        return 0

    lax.fori_loop(0, NCH, chunk_body, 0)
    write_wait(0 if NCH % 2 == 1 else 1)
    write_wait(1 if NCH % 2 == 1 else 0)


@functools.partial(jax.jit, static_argnames=())
def kernel(time, dis, speed, W_time, W_dis, W_speed):
    def tiled(x):
        return (x.astype(jnp.int32).T
                .reshape(L // 8, 8, NW, BT).transpose(0, 2, 1, 3))

    t = tiled(time)
    d = tiled(dis)
    s = tiled(speed)

    mesh = plsc.VectorSubcoreMesh(core_axis_name="c", subcore_axis_name="s")
    run = pl.kernel(
        _body,
        out_type=jax.ShapeDtypeStruct((L, 3, NW, EMBED, BT), jnp.float32),
        mesh=mesh,
        scratch_types=[
            pltpu.VMEM((2, LC, BT), jnp.int32),
            pltpu.VMEM((2, M), jnp.int32),
            pltpu.VMEM((2, LC, BT), jnp.int32),
            pltpu.VMEM((2, M, EMBED), jnp.float32),
            pltpu.VMEM((2, LC, 3, EMBED, BT), jnp.float32),
            pltpu.VMEM((EMBED, TIME_V), jnp.float32),
            pltpu.VMEM((EMBED, SPEED_V), jnp.float32),
            pltpu.SemaphoreType.DMA,
            pltpu.SemaphoreType.DMA,
            pltpu.SemaphoreType.DMA,
        ],
        compiler_params=pltpu.CompilerParams(
            needs_layout_passes=False,
            use_tc_tiling_on_sc=False,
        ),
    )
    out5 = run(t, d, s, W_time.T, W_dis, W_speed.T)
    return out5.transpose(2, 4, 0, 1, 3).reshape(B, L, OUT_D)
